# Initial kernel scaffold; baseline (speedup 1.0000x reference)
import jax
import jax.numpy as jnp
from jax.experimental import pallas as pl


def kernel(x, edge_index, W1_l, W1_r, b1, W2_l, W2_r, b2):
    raise NotImplementedError("write your pallas kernel here")


# speedup vs baseline: 5.2942x; 5.2942x over previous
import functools

import jax
import jax.numpy as jnp
from jax import lax
from jax.experimental import pallas as pl
from jax.experimental.pallas import tpu as pltpu
from jax.experimental.pallas import tpu_sc as plsc

N = 10000
E = 320000
D = 128

NC = 2
NS = 16
NW = NC * NS
EPW = E // NW
CHUNK = 80
NCHUNK = EPW // CHUNK
NP = 10240
RPT = NP // NS

_mesh = plsc.VectorSubcoreMesh(
    core_axis_name="c", subcore_axis_name="s", num_cores=NC, num_subcores=NS
)


def _agg_body(with_cnt, *refs):
    if with_cnt:
        (x_hbm, src_hbm, dst_hbm, zacc_hbm,
         acc_out, cnt_out,
         acc_sh, cnt_sh, sidx_v, didx_v, rows_v, ones_v, cv, sem) = refs
    else:
        (x_hbm, src_hbm, dst_hbm, zacc_hbm,
         acc_out,
         acc_sh, sidx_v, didx_v, rows_v, sem) = refs

    cid = lax.axis_index("c")
    sid = lax.axis_index("s")
    wid = cid * NS + sid

    r0 = pl.multiple_of(sid * RPT, 8)
    pltpu.sync_copy(zacc_hbm.at[pl.ds(r0, RPT)], acc_sh.at[pl.ds(r0, RPT)])
    if with_cnt:
        z16 = jnp.zeros((16,), jnp.float32)
        o16 = jnp.ones((16,), jnp.float32)

        def zrow(r, c):
            cv[pl.ds(r * 16, 16)] = z16
            return c

        lax.fori_loop(0, RPT // 16, zrow, 0)

        def orow(r, c):
            ones_v[pl.ds(r * 16, 16)] = o16
            return c

        lax.fori_loop(0, CHUNK // 16, orow, 0)
        pltpu.sync_copy(cv, cnt_sh.at[pl.ds(r0, RPT)])
    plsc.subcore_barrier()

    base0 = wid * EPW

    def chunk_step(i, carry):
        base = pl.multiple_of(base0 + i * CHUNK, 8)
        pltpu.sync_copy(src_hbm.at[pl.ds(base, CHUNK)], sidx_v)
        pltpu.sync_copy(dst_hbm.at[pl.ds(base, CHUNK)], didx_v)
        pltpu.async_copy(x_hbm.at[sidx_v], rows_v, sem).wait()
        pltpu.sync_copy(rows_v, acc_sh.at[didx_v], add=True)
        if with_cnt:
            pltpu.sync_copy(ones_v, cnt_sh.at[didx_v], add=True)
        return carry

    lax.fori_loop(0, NCHUNK, chunk_step, 0)
    plsc.subcore_barrier()

    pltpu.sync_copy(acc_sh.at[pl.ds(r0, RPT)], acc_out.at[cid, pl.ds(r0, RPT)])
    if with_cnt:
        pltpu.sync_copy(cnt_sh.at[pl.ds(r0, RPT)], cv)
        pltpu.sync_copy(cv, cnt_out.at[pl.ds(cid * NP + r0, RPT)])


_agg_with_cnt = pl.kernel(
    functools.partial(_agg_body, True),
    out_type=(
        jax.ShapeDtypeStruct((NC, NP, D), jnp.float32),
        jax.ShapeDtypeStruct((NC * NP,), jnp.float32),
    ),
    mesh=_mesh,
    scratch_types=[
        pltpu.VMEM_SHARED((NP, D), jnp.float32),
        pltpu.VMEM_SHARED((NP,), jnp.float32),
        pltpu.VMEM((CHUNK,), jnp.int32),
        pltpu.VMEM((CHUNK,), jnp.int32),
        pltpu.VMEM((CHUNK, D), jnp.float32),
        pltpu.VMEM((CHUNK,), jnp.float32),
        pltpu.VMEM((RPT,), jnp.float32),
        pltpu.SemaphoreType.DMA,
    ],
    name="sage_agg_cnt",
)

_agg_no_cnt = pl.kernel(
    functools.partial(_agg_body, False),
    out_type=jax.ShapeDtypeStruct((NC, NP, D), jnp.float32),
    mesh=_mesh,
    scratch_types=[
        pltpu.VMEM_SHARED((NP, D), jnp.float32),
        pltpu.VMEM((CHUNK,), jnp.int32),
        pltpu.VMEM((CHUNK,), jnp.int32),
        pltpu.VMEM((CHUNK, D), jnp.float32),
        pltpu.SemaphoreType.DMA,
    ],
    name="sage_agg",
)

BN = 400


def _combine_body(p_ref, c_ref, x_ref, wl_ref, wr_ref, b_ref, o_ref):
    cnt = jnp.maximum(c_ref[0] + c_ref[1], 1.0)
    agg = (p_ref[0] + p_ref[1]) / cnt
    acc = jax.lax.dot_general(
        agg, wl_ref[...], (((1,), (0,)), ((), ())),
        preferred_element_type=jnp.float32,
        precision=jax.lax.Precision.HIGHEST)
    acc = acc + jax.lax.dot_general(
        x_ref[...], wr_ref[...], (((1,), (0,)), ((), ())),
        preferred_element_type=jnp.float32,
        precision=jax.lax.Precision.HIGHEST)
    o_ref[...] = jnp.maximum(acc + b_ref[...], 0.0)


def _combine(p, c, x, W_l, W_r, b):
    return pl.pallas_call(
        _combine_body,
        grid=(N // BN,),
        in_specs=[
            pl.BlockSpec((NC, BN, D), lambda i: (0, i, 0)),
            pl.BlockSpec((NC, BN, 1), lambda i: (0, i, 0)),
            pl.BlockSpec((BN, D), lambda i: (i, 0)),
            pl.BlockSpec((D, D), lambda i: (0, 0)),
            pl.BlockSpec((D, D), lambda i: (0, 0)),
            pl.BlockSpec((1, D), lambda i: (0, 0)),
        ],
        out_specs=pl.BlockSpec((BN, D), lambda i: (i, 0)),
        out_shape=jax.ShapeDtypeStruct((N, D), jnp.float32),
    )(p, c, x, W_l, W_r, b)


@jax.jit
def kernel(x, edge_index, W1_l, W1_r, b1, W2_l, W2_r, b2):
    src = edge_index[0]
    dst = edge_index[1]
    zacc = jnp.zeros((NP, D), jnp.float32)
    b1r = b1.reshape(1, D)
    b2r = b2.reshape(1, D)

    p1, cnt_flat = _agg_with_cnt(x, src, dst, zacc)
    cnt = cnt_flat.reshape(NC, NP, 1)
    h = _combine(p1, cnt, x, W1_l, W1_r, b1r)
    p2 = _agg_no_cnt(h, src, dst, zacc)
    return _combine(p2, cnt, h, W2_l, W2_r, b2r)
